# baseline (speedup 1.0000x reference)
import functools

import jax
import jax.numpy as jnp
from jax import lax
from jax.experimental import pallas as pl
from jax.experimental.pallas import tpu as pltpu
from jax.experimental.pallas import tpu_sc as plsc

F = 26
V1 = 100001
E = 16
B = 16384

NC = 2
NS = 16
NW = NC * NS

TPW = (F * E) // NW
OCH = 4096
NOCH = B // OCH
GPC = OCH // 16
VA = 99968
VT = V1 - VA

_mesh = plsc.VectorSubcoreMesh(core_axis_name="c", subcore_axis_name="s")


@functools.partial(
    pl.kernel,
    mesh=_mesh,
    out_type=jax.ShapeDtypeStruct((F, E, B), jnp.float32),
    scratch_types=[
        pltpu.VMEM((VA + 128,), jnp.float32),
        pltpu.VMEM((B,), jnp.int32),
        pltpu.VMEM((OCH,), jnp.float32),
        pltpu.VMEM((OCH,), jnp.float32),
        pltpu.SemaphoreType.DMA,
        pltpu.SemaphoreType.DMA,
        pltpu.SemaphoreType.DMA,
    ],
    compiler_params=pltpu.CompilerParams(needs_layout_passes=False),
)
def _sc_lookup(idx_hbm, tab_hbm, tails_hbm, out_hbm, tab_v, idx_v, out0, out1,
               tsem, w0sem, w1sem):
    wid = lax.axis_index("s") * NC + lax.axis_index("c")
    obufs = (out0, out1)
    wsems = (w0sem, w1sem)
    prev_wb = [None, None]
    for j in range(TPW):
        t = wid * TPW + j
        f = t // E
        e = t % E
        tcopies = [
            pltpu.async_copy(
                tab_hbm.at[f, e, pl.ds(off, ln)],
                tab_v.at[pl.ds(off, ln)],
                tsem)
            for off, ln in ((0, 33280), (33280, 33280), (66560, 33408))
        ]
        tcopies.append(
            pltpu.async_copy(tails_hbm.at[f, e], tab_v.at[pl.ds(VA, 128)],
                             tsem))
        if j == 0:
            pltpu.sync_copy(idx_hbm.at[f], idx_v)
        else:
            @pl.when(e == 0)
            def _():
                pltpu.sync_copy(idx_hbm.at[f], idx_v)
        for tc_ in tcopies:
            tc_.wait()
        for ch in range(NOCH):
            b = ch % 2
            if prev_wb[b] is not None:
                prev_wb[b].wait()
            ob = obufs[b]

            @plsc.parallel_loop(0, GPC, unroll=8)
            def _gather(i, ch=ch, ob=ob):
                idx16 = idx_v[pl.ds(ch * OCH + i * 16, 16)]
                ob[pl.ds(i * 16, 16)] = plsc.load_gather(tab_v, [idx16])

            prev_wb[b] = pltpu.async_copy(
                ob, out_hbm.at[f, e, pl.ds(ch * OCH, OCH)], wsems[b])
    prev_wb[0].wait()
    prev_wb[1].wait()


def kernel(indices, tables):
    idx_t = indices.T
    tab_t = jnp.transpose(tables, (0, 2, 1))
    tails = lax.slice(tab_t, (0, 0, VA), (F, E, V1))
    tails = jnp.pad(tails, ((0, 0), (0, 0), (0, 128 - VT)))
    out_t = _sc_lookup(idx_t, tab_t, tails)
    return jnp.transpose(out_t, (2, 0, 1))


# speedup vs baseline: 1.0063x; 1.0063x over previous
import functools

import jax
import jax.numpy as jnp
from jax import lax
from jax.experimental import pallas as pl
from jax.experimental.pallas import tpu as pltpu
from jax.experimental.pallas import tpu_sc as plsc

F = 26
V1 = 100001
E = 16
B = 16384

NC = 2
NS = 16
NW = NC * NS

TPW = (F * E) // NW
OCH = 4096
NOCH = B // OCH
GPC = OCH // 16

_mesh = plsc.VectorSubcoreMesh(core_axis_name="c", subcore_axis_name="s")


@functools.partial(
    pl.kernel,
    mesh=_mesh,
    out_type=jax.ShapeDtypeStruct((F, E, B), jnp.float32),
    scratch_types=[
        pltpu.VMEM((V1,), jnp.float32),
        pltpu.VMEM((B,), jnp.int32),
        pltpu.VMEM((OCH,), jnp.float32),
        pltpu.VMEM((OCH,), jnp.float32),
        pltpu.SemaphoreType.DMA,
        pltpu.SemaphoreType.DMA,
        pltpu.SemaphoreType.DMA,
    ],
    compiler_params=pltpu.CompilerParams(needs_layout_passes=False),
)
def _sc_lookup(idx_hbm, tab_hbm, out_hbm, tab_v, idx_v, out0, out1,
               tsem, w0sem, w1sem):
    wid = lax.axis_index("s") * NC + lax.axis_index("c")
    obufs = (out0, out1)
    wsems = (w0sem, w1sem)
    prev_wb = [None, None]
    for j in range(TPW):
        t = wid * TPW + j
        f = t // E
        e = t % E
        tcopy = pltpu.async_copy(tab_hbm.at[f, e], tab_v, tsem)
        if j == 0:
            pltpu.sync_copy(idx_hbm.at[f], idx_v)
        else:
            @pl.when(e == 0)
            def _():
                pltpu.sync_copy(idx_hbm.at[f], idx_v)
        tcopy.wait()
        for ch in range(NOCH):
            b = ch % 2
            if prev_wb[b] is not None:
                prev_wb[b].wait()
            ob = obufs[b]

            @plsc.parallel_loop(0, GPC, unroll=8)
            def _gather(i, ch=ch, ob=ob):
                idx16 = idx_v[pl.ds(ch * OCH + i * 16, 16)]
                ob[pl.ds(i * 16, 16)] = plsc.load_gather(tab_v, [idx16])

            prev_wb[b] = pltpu.async_copy(
                ob, out_hbm.at[f, e, pl.ds(ch * OCH, OCH)], wsems[b])
    prev_wb[0].wait()
    prev_wb[1].wait()


def kernel(indices, tables):
    idx_t = indices.T
    tab_t = jnp.transpose(tables, (0, 2, 1))
    out_t = _sc_lookup(idx_t, tab_t)
    return jnp.transpose(out_t, (2, 0, 1))
